# baseline (speedup 1.0000x reference)
import functools

import jax
import jax.numpy as jnp
from jax import lax
from jax.experimental import pallas as pl
from jax.experimental.pallas import tpu as pltpu
from jax.experimental.pallas import tpu_sc as plsc

_B = 16384
_D = 128
_C = 1000
_NC = 2
_NS = 16
_NW = _NC * _NS
_RPW = _B // _NW
_CH = 64
_NCH = _RPW // _CH
_LANES = 16
_JV = _D // _LANES
_CPT = 64
_ADD_LAG = 2


def _sc_body(latent_hbm, labels_hbm, centers_hbm, out_hbm, lab_v,
             stage_v, res_v, neg_sh, out_sem, *bufs_and_sems):
    bufs = bufs_and_sems[:_NCH]
    fill_sems = bufs_and_sems[_NCH:2 * _NCH]
    add_sems = bufs_and_sems[2 * _NCH:]
    sid = lax.axis_index("s")
    wid = sid * _NC + lax.axis_index("c")
    pltpu.sync_copy(labels_hbm.at[wid], lab_v)

    base = jnp.minimum(sid * _CPT, _C - _CPT)
    pltpu.sync_copy(centers_hbm.at[pl.ds(base, _CPT)], stage_v)

    @plsc.parallel_loop(0, _CPT, 1)
    def _neg_loop(r):
        for j in range(_JV):
            stage_v[r, pl.ds(j * _LANES, _LANES)] = (
                -stage_v[r, pl.ds(j * _LANES, _LANES)])

    pltpu.sync_copy(stage_v, neg_sh.at[pl.ds(base, _CPT)])

    def fill(ch):
        row0 = wid * _RPW + ch * _CH
        return pltpu.async_copy(
            latent_hbm.at[pl.ds(row0, _CH)], bufs[ch], fill_sems[ch])

    def gather_add(ch):
        return pltpu.async_copy(
            neg_sh.at[lab_v.at[ch]], bufs[ch], add_sems[ch], add=True)

    accs = tuple(jnp.zeros((_LANES,), jnp.float32) for _ in range(_JV))

    fills = [fill(ch) for ch in range(_NCH)]
    plsc.subcore_barrier()
    adds = {}
    for ch in range(_ADD_LAG):
        fills[ch].wait()
        adds[ch] = gather_add(ch)
    for ch in range(_NCH):
        adds[ch].wait()
        buf = bufs[ch]

        @plsc.parallel_loop(0, _CH, 1, unroll=4, carry=accs)
        def row_loop(r, acc_in):
            new = []
            for j in range(_JV):
                d = buf[r, pl.ds(j * _LANES, _LANES)]
                new.append(acc_in[j] + d * d)
            return tuple(new)

        accs = row_loop
        nxt = ch + _ADD_LAG
        if nxt < _NCH:
            fills[nxt].wait()
            adds[nxt] = gather_add(nxt)

    total = accs[0]
    for j in range(1, _JV):
        total = total + accs[j]
    res_v[...] = total
    pltpu.async_copy(res_v, out_hbm.at[wid], out_sem).wait()


@jax.jit
def _center_loss_partials(latent, labels3d, centers):
    mesh = plsc.VectorSubcoreMesh(core_axis_name="c", subcore_axis_name="s")
    run = functools.partial(
        pl.kernel,
        out_type=jax.ShapeDtypeStruct((_NW, _LANES), jnp.float32),
        mesh=mesh,
        scratch_types=(
            [
                pltpu.VMEM((_NCH, _CH), jnp.int32),
                pltpu.VMEM((_CPT, _D), jnp.float32),
                pltpu.VMEM((_LANES,), jnp.float32),
                pltpu.VMEM_SHARED((_C, _D), jnp.float32),
                pltpu.SemaphoreType.DMA,
            ]
            + [pltpu.VMEM((_CH, _D), jnp.float32) for _ in range(_NCH)]
            + [pltpu.SemaphoreType.DMA for _ in range(2 * _NCH)]
        ),
    )(_sc_body)
    return run(latent, labels3d, centers)


def kernel(latent, labels, centers):
    labels3d = labels.astype(jnp.int32).reshape(_NW, _NCH, _CH)
    partials = _center_loss_partials(latent, labels3d, centers)
    return jnp.sum(partials) / jnp.float32(_B)


# speedup vs baseline: 1.0041x; 1.0041x over previous
import functools

import jax
import jax.numpy as jnp
from jax import lax
from jax.experimental import pallas as pl
from jax.experimental.pallas import tpu as pltpu
from jax.experimental.pallas import tpu_sc as plsc

_B = 16384
_D = 128
_C = 1000
_NC = 2
_NS = 16
_NW = _NC * _NS
_RPW = _B // _NW
_CH = 64
_NCH = _RPW // _CH
_LANES = 16
_JV = _D // _LANES
_CPT = 64
_ADD_LAG = 2


def _sc_body(latent_hbm, labels_hbm, centers_hbm, out_hbm, lab_v,
             stage_v, res_v, neg_sh, out_sem, *bufs_and_sems):
    bufs = bufs_and_sems[:_NCH]
    fill_sems = bufs_and_sems[_NCH:2 * _NCH]
    add_sems = bufs_and_sems[2 * _NCH:]
    sid = lax.axis_index("s")
    wid = sid * _NC + lax.axis_index("c")
    pltpu.sync_copy(labels_hbm.at[pl.ds(wid * _RPW, _RPW)], lab_v)

    base = jnp.minimum(sid * _CPT, _C - _CPT)
    pltpu.sync_copy(centers_hbm.at[pl.ds(base, _CPT)], stage_v)

    @plsc.parallel_loop(0, _CPT, 1)
    def _neg_loop(r):
        for j in range(_JV):
            stage_v[r, pl.ds(j * _LANES, _LANES)] = (
                -stage_v[r, pl.ds(j * _LANES, _LANES)])

    pltpu.sync_copy(stage_v, neg_sh.at[pl.ds(base, _CPT)])

    def fill(ch):
        row0 = wid * _RPW + ch * _CH
        return pltpu.async_copy(
            latent_hbm.at[pl.ds(row0, _CH)], bufs[ch], fill_sems[ch])

    def gather_add(ch):
        idx = lab_v.at[pl.ds(ch * _CH, _CH)]
        return pltpu.async_copy(
            neg_sh.at[idx], bufs[ch], add_sems[ch], add=True)

    accs = tuple(jnp.zeros((_LANES,), jnp.float32) for _ in range(_JV))

    fills = [fill(ch) for ch in range(_NCH)]
    plsc.subcore_barrier()
    adds = {}
    for ch in range(_ADD_LAG):
        fills[ch].wait()
        adds[ch] = gather_add(ch)
    for ch in range(_NCH):
        adds[ch].wait()
        buf = bufs[ch]

        @plsc.parallel_loop(0, _CH, 1, unroll=4, carry=accs)
        def row_loop(r, acc_in):
            new = []
            for j in range(_JV):
                d = buf[r, pl.ds(j * _LANES, _LANES)]
                new.append(acc_in[j] + d * d)
            return tuple(new)

        accs = row_loop
        nxt = ch + _ADD_LAG
        if nxt < _NCH:
            fills[nxt].wait()
            adds[nxt] = gather_add(nxt)

    total = accs[0]
    for j in range(1, _JV):
        total = total + accs[j]
    res_v[...] = total
    pltpu.async_copy(res_v, out_hbm.at[wid], out_sem).wait()


@jax.jit
def _center_loss_partials(latent, labels1d, centers):
    mesh = plsc.VectorSubcoreMesh(core_axis_name="c", subcore_axis_name="s")
    run = functools.partial(
        pl.kernel,
        out_type=jax.ShapeDtypeStruct((_NW, _LANES), jnp.float32),
        mesh=mesh,
        scratch_types=(
            [
                pltpu.VMEM((_RPW,), jnp.int32),
                pltpu.VMEM((_CPT, _D), jnp.float32),
                pltpu.VMEM((_LANES,), jnp.float32),
                pltpu.VMEM_SHARED((_C, _D), jnp.float32),
                pltpu.SemaphoreType.DMA,
            ]
            + [pltpu.VMEM((_CH, _D), jnp.float32) for _ in range(_NCH)]
            + [pltpu.SemaphoreType.DMA for _ in range(2 * _NCH)]
        ),
    )(_sc_body)
    return run(latent, labels1d, centers)


def kernel(latent, labels, centers):
    partials = _center_loss_partials(latent, labels.astype(jnp.int32), centers)
    return jnp.sum(partials) / jnp.float32(_B)


# speedup vs baseline: 1.0147x; 1.0105x over previous
import functools

import jax
import jax.numpy as jnp
from jax import lax
from jax.experimental import pallas as pl
from jax.experimental.pallas import tpu as pltpu
from jax.experimental.pallas import tpu_sc as plsc

_B = 16384
_D = 128
_C = 1000
_NC = 2
_NS = 16
_NW = _NC * _NS
_RPW = _B // _NW
_CH = 64
_NCH = _RPW // _CH
_NBUF = 4
_LANES = 16
_JV = _D // _LANES
_CPT = 64


def _sc_body(latent_hbm, labels_hbm, centers_hbm, out_hbm, lab_v,
             res_v, cen_sh, out_sem, *bufs_and_sems):
    lat_bufs = bufs_and_sems[:_NBUF]
    cen_bufs = bufs_and_sems[_NBUF:2 * _NBUF]
    fill_sems = bufs_and_sems[2 * _NBUF:3 * _NBUF]
    gat_sems = bufs_and_sems[3 * _NBUF:]
    sid = lax.axis_index("s")
    wid = sid * _NC + lax.axis_index("c")
    pltpu.sync_copy(labels_hbm.at[pl.ds(wid * _RPW, _RPW)], lab_v)

    base = jnp.minimum(sid * _CPT, _C - _CPT)
    pltpu.sync_copy(centers_hbm.at[pl.ds(base, _CPT)],
                    cen_sh.at[pl.ds(base, _CPT)])

    def start(ch):
        b = ch % _NBUF
        row0 = wid * _RPW + ch * _CH
        f = pltpu.async_copy(
            latent_hbm.at[pl.ds(row0, _CH)], lat_bufs[b], fill_sems[b])
        g = pltpu.async_copy(
            cen_sh.at[lab_v.at[pl.ds(ch * _CH, _CH)]], cen_bufs[b],
            gat_sems[b])
        return f, g

    accs = tuple(jnp.zeros((_LANES,), jnp.float32) for _ in range(_JV))

    pend = {}
    plsc.subcore_barrier()
    for ch in range(_NBUF):
        pend[ch] = start(ch)
    for ch in range(_NCH):
        b = ch % _NBUF
        f, g = pend.pop(ch)
        f.wait()
        g.wait()
        lat_v, cen_v = lat_bufs[b], cen_bufs[b]

        @plsc.parallel_loop(0, _CH, 1, unroll=4, carry=accs)
        def row_loop(r, acc_in):
            new = []
            for j in range(_JV):
                lt = lat_v[r, pl.ds(j * _LANES, _LANES)]
                cn = cen_v[r, pl.ds(j * _LANES, _LANES)]
                d = lt - cn
                new.append(acc_in[j] + d * d)
            return tuple(new)

        accs = row_loop
        if ch + _NBUF < _NCH:
            pend[ch + _NBUF] = start(ch + _NBUF)

    total = accs[0]
    for j in range(1, _JV):
        total = total + accs[j]
    res_v[...] = total
    pltpu.async_copy(res_v, out_hbm.at[wid], out_sem).wait()


@jax.jit
def _center_loss_partials(latent, labels1d, centers):
    mesh = plsc.VectorSubcoreMesh(core_axis_name="c", subcore_axis_name="s")
    run = functools.partial(
        pl.kernel,
        out_type=jax.ShapeDtypeStruct((_NW, _LANES), jnp.float32),
        mesh=mesh,
        scratch_types=(
            [
                pltpu.VMEM((_RPW,), jnp.int32),
                pltpu.VMEM((_LANES,), jnp.float32),
                pltpu.VMEM_SHARED((_C, _D), jnp.float32),
                pltpu.SemaphoreType.DMA,
            ]
            + [pltpu.VMEM((_CH, _D), jnp.float32) for _ in range(2 * _NBUF)]
            + [pltpu.SemaphoreType.DMA for _ in range(2 * _NBUF)]
        ),
    )(_sc_body)
    return run(latent, labels1d, centers)


def kernel(latent, labels, centers):
    partials = _center_loss_partials(latent, labels.astype(jnp.int32), centers)
    return jnp.sum(partials) / jnp.float32(_B)


# speedup vs baseline: 1.0391x; 1.0241x over previous
import functools

import jax
import jax.numpy as jnp
from jax import lax
from jax.experimental import pallas as pl
from jax.experimental.pallas import tpu as pltpu
from jax.experimental.pallas import tpu_sc as plsc

_B = 16384
_D = 128
_C = 1000
_NC = 2
_NS = 16
_NW = _NC * _NS
_RPW = _B // _NW
_CH = 128
_NCH = _RPW // _CH
_NBUF = 3
_LANES = 16
_JV = _D // _LANES
_CPT = 64


def _sc_body(latent_hbm, labels_hbm, centers_hbm, out_hbm, lab_v,
             res_v, cen_sh, out_sem, *bufs_and_sems):
    lat_bufs = bufs_and_sems[:_NBUF]
    cen_bufs = bufs_and_sems[_NBUF:2 * _NBUF]
    fill_sems = bufs_and_sems[2 * _NBUF:3 * _NBUF]
    gat_sems = bufs_and_sems[3 * _NBUF:]
    sid = lax.axis_index("s")
    wid = sid * _NC + lax.axis_index("c")

    def fill(ch):
        b = ch % _NBUF
        row0 = wid * _RPW + ch * _CH
        return pltpu.async_copy(
            latent_hbm.at[pl.ds(row0, _CH)], lat_bufs[b], fill_sems[b])

    def gather(ch):
        b = ch % _NBUF
        return pltpu.async_copy(
            cen_sh.at[lab_v.at[pl.ds(ch * _CH, _CH)]], cen_bufs[b],
            gat_sems[b])

    fills = {ch: fill(ch) for ch in range(_NBUF)}

    pltpu.sync_copy(labels_hbm.at[pl.ds(wid * _RPW, _RPW)], lab_v)

    base = jnp.minimum(sid * _CPT, _C - _CPT)
    pltpu.sync_copy(centers_hbm.at[pl.ds(base, _CPT)],
                    cen_sh.at[pl.ds(base, _CPT)])

    accs = tuple(jnp.zeros((_LANES,), jnp.float32) for _ in range(_JV))

    plsc.subcore_barrier()
    gats = {ch: gather(ch) for ch in range(_NBUF)}
    for ch in range(_NCH):
        b = ch % _NBUF
        fills.pop(ch).wait()
        gats.pop(ch).wait()
        lat_v, cen_v = lat_bufs[b], cen_bufs[b]

        @plsc.parallel_loop(0, _CH, 1, unroll=4, carry=accs)
        def row_loop(r, acc_in):
            new = []
            for j in range(_JV):
                lt = lat_v[r, pl.ds(j * _LANES, _LANES)]
                cn = cen_v[r, pl.ds(j * _LANES, _LANES)]
                d = lt - cn
                new.append(acc_in[j] + d * d)
            return tuple(new)

        accs = row_loop
        if ch + _NBUF < _NCH:
            fills[ch + _NBUF] = fill(ch + _NBUF)
            gats[ch + _NBUF] = gather(ch + _NBUF)

    total = accs[0]
    for j in range(1, _JV):
        total = total + accs[j]
    res_v[...] = total
    pltpu.async_copy(res_v, out_hbm.at[wid], out_sem).wait()


@jax.jit
def _center_loss_partials(latent, labels1d, centers):
    mesh = plsc.VectorSubcoreMesh(core_axis_name="c", subcore_axis_name="s")
    run = functools.partial(
        pl.kernel,
        out_type=jax.ShapeDtypeStruct((_NW, _LANES), jnp.float32),
        mesh=mesh,
        scratch_types=(
            [
                pltpu.VMEM((_RPW,), jnp.int32),
                pltpu.VMEM((_LANES,), jnp.float32),
                pltpu.VMEM_SHARED((_C, _D), jnp.float32),
                pltpu.SemaphoreType.DMA,
            ]
            + [pltpu.VMEM((_CH, _D), jnp.float32) for _ in range(2 * _NBUF)]
            + [pltpu.SemaphoreType.DMA for _ in range(2 * _NBUF)]
        ),
    )(_sc_body)
    return run(latent, labels1d, centers)


def kernel(latent, labels, centers):
    partials = _center_loss_partials(latent, labels.astype(jnp.int32), centers)
    return jnp.sum(partials) / jnp.float32(_B)


# speedup vs baseline: 1.0638x; 1.0237x over previous
import functools

import jax
import jax.numpy as jnp
from jax import lax
from jax.experimental import pallas as pl
from jax.experimental.pallas import tpu as pltpu
from jax.experimental.pallas import tpu_sc as plsc

_B = 16384
_D = 128
_C = 1000
_NC = 2
_NS = 16
_NW = _NC * _NS
_RPW = _B // _NW
_CH = 128
_NCH = _RPW // _CH
_NBUF = 3
_LANES = 16
_JV = _D // _LANES
_CPT = 64


def _sc_body(latent_hbm, labels_hbm, centers_hbm, out_hbm, lab_v,
             res_v, cen_sh, out_sem, stage_sem, *bufs_and_sems):
    lat_bufs = bufs_and_sems[:_NBUF]
    cen_bufs = bufs_and_sems[_NBUF:2 * _NBUF]
    fill_sems = bufs_and_sems[2 * _NBUF:3 * _NBUF]
    gat_sems = bufs_and_sems[3 * _NBUF:]
    sid = lax.axis_index("s")
    wid = sid * _NC + lax.axis_index("c")

    def fill(ch):
        b = ch % _NBUF
        row0 = wid * _RPW + ch * _CH
        return pltpu.async_copy(
            latent_hbm.at[pl.ds(row0, _CH)], lat_bufs[b], fill_sems[b])

    def gather(ch):
        b = ch % _NBUF
        return pltpu.async_copy(
            cen_sh.at[lab_v.at[pl.ds(ch * _CH, _CH)]], cen_bufs[b],
            gat_sems[b])

    fills = {ch: fill(ch) for ch in range(_NBUF)}

    base = jnp.minimum(sid * _CPT, _C - _CPT)
    lab_copy = pltpu.async_copy(
        labels_hbm.at[pl.ds(wid * _RPW, _RPW)], lab_v, stage_sem)
    stg_copy = pltpu.async_copy(
        centers_hbm.at[pl.ds(base, _CPT)], cen_sh.at[pl.ds(base, _CPT)],
        stage_sem)

    accs = tuple(jnp.zeros((_LANES,), jnp.float32) for _ in range(_JV))

    lab_copy.wait()
    stg_copy.wait()
    plsc.subcore_barrier()
    gats = {ch: gather(ch) for ch in range(_NBUF)}
    for ch in range(_NCH):
        b = ch % _NBUF
        fills.pop(ch).wait()
        gats.pop(ch).wait()
        lat_v, cen_v = lat_bufs[b], cen_bufs[b]

        @plsc.parallel_loop(0, _CH, 1, unroll=4, carry=accs)
        def row_loop(r, acc_in):
            new = []
            for j in range(_JV):
                lt = lat_v[r, pl.ds(j * _LANES, _LANES)]
                cn = cen_v[r, pl.ds(j * _LANES, _LANES)]
                d = lt - cn
                new.append(acc_in[j] + d * d)
            return tuple(new)

        accs = row_loop
        if ch + _NBUF < _NCH:
            fills[ch + _NBUF] = fill(ch + _NBUF)
            gats[ch + _NBUF] = gather(ch + _NBUF)

    total = accs[0]
    for j in range(1, _JV):
        total = total + accs[j]
    res_v[...] = total
    pltpu.async_copy(res_v, out_hbm.at[wid], out_sem).wait()


@jax.jit
def _center_loss_partials(latent, labels1d, centers):
    mesh = plsc.VectorSubcoreMesh(core_axis_name="c", subcore_axis_name="s")
    run = functools.partial(
        pl.kernel,
        out_type=jax.ShapeDtypeStruct((_NW, _LANES), jnp.float32),
        mesh=mesh,
        scratch_types=(
            [
                pltpu.VMEM((_RPW,), jnp.int32),
                pltpu.VMEM((_LANES,), jnp.float32),
                pltpu.VMEM_SHARED((_C, _D), jnp.float32),
                pltpu.SemaphoreType.DMA,
                pltpu.SemaphoreType.DMA,
            ]
            + [pltpu.VMEM((_CH, _D), jnp.float32) for _ in range(2 * _NBUF)]
            + [pltpu.SemaphoreType.DMA for _ in range(2 * _NBUF)]
        ),
    )(_sc_body)
    return run(latent, labels1d, centers)


def kernel(latent, labels, centers):
    partials = _center_loss_partials(latent, labels.astype(jnp.int32), centers)
    return jnp.sum(partials) / jnp.float32(_B)
